# baseline (speedup 1.0000x reference)
import functools

import jax
import jax.numpy as jnp
from jax.experimental import pallas as pl
from jax.experimental.pallas import tpu as pltpu
from jax.experimental.pallas import tpu_sc as plsc

_B = 16
_N = 32768
_NF = 32
_C1 = 64
_C2 = 16
_K = 64
_CN = 4096
_EPS = 1e-5
_BIGI = 2 ** 30


def _first_step():
    return (pl.program_id(0) == 0) & (pl.program_id(1) == 0)


def _stats_cols(x):
    ssum = jnp.sum(x, axis=1, keepdims=True)
    ssq = jnp.sum(x * x, axis=1, keepdims=True)
    lane = jax.lax.broadcasted_iota(jnp.int32, (x.shape[0], 128), 1)
    return jnp.where(lane == 0, ssum, 0.0) + jnp.where(lane == 1, ssq, 0.0)


def _mm(w, x):
    return jax.lax.dot_general(w, x, (((1,), (0,)), ((), ())),
                               preferred_element_type=jnp.float32)


def _stats_kernel_a(s_ref, w_ref, b_ref, out_ref, sbf_ref):
    sb = s_ref[0].astype(jnp.bfloat16)
    sbf_ref[0] = sb
    x = _mm(w_ref[...], sb) + b_ref[...]

    @pl.when(_first_step())
    def _():
        out_ref[...] = jnp.zeros_like(out_ref)

    out_ref[...] += _stats_cols(x)


def _layer12(s_ref, w0_ref, aff0_ref, w1_ref, b1_ref):
    x0 = _mm(w0_ref[...], s_ref[0]) + aff0_ref[:, 2:3]
    x1 = jnp.maximum(aff0_ref[:, 0:1] * x0 + aff0_ref[:, 1:2], 0.0)
    return _mm(w1_ref[...], x1.astype(jnp.bfloat16)) + b1_ref[...]


def _stats_kernel_b(s_ref, w0_ref, aff0_ref, w1_ref, b1_ref, out_ref):
    t = _layer12(s_ref, w0_ref, aff0_ref, w1_ref, b1_ref)

    @pl.when(_first_step())
    def _():
        out_ref[...] = jnp.zeros_like(out_ref)

    out_ref[...] += _stats_cols(t)


def _score_kernel_c(s_ref, w0_ref, aff0_ref, w1_ref, b1_ref, aff1_ref,
                    w2_ref, b2_ref, u_ref, us_ref):
    t = _layer12(s_ref, w0_ref, aff0_ref, w1_ref, b1_ref)
    x2 = jnp.maximum(aff1_ref[:, 0:1] * t + aff1_ref[:, 1:2], 0.0)
    u = _mm(w2_ref[...], x2.astype(jnp.bfloat16))
    uf = u[0] + b2_ref[0]
    u_ref[0, 0, :] = uf
    r_io = jax.lax.broadcasted_iota(jnp.int32, (8, 128), 0)
    l_io = jax.lax.broadcasted_iota(jnp.int32, (8, 128), 1)
    contrib = (jnp.where((r_io == 0) & (l_io == 0), jnp.sum(uf), 0.0)
               + jnp.where((r_io == 1) & (l_io == 0), jnp.sum(uf * uf), 0.0))

    @pl.when(_first_step())
    def _():
        us_ref[...] = jnp.zeros_like(us_ref)

    us_ref[...] += contrib


def _sc_select(u_hbm, scal_hbm, vals_hbm, idx_hbm, sv_v, cand_v, scal_v,
               ov_v, oi_v):
    wid = jax.lax.axis_index("s") * 2 + jax.lax.axis_index("c")

    @pl.when(wid < _B)
    def _():
        pltpu.sync_copy(scal_hbm, scal_v)
        pltpu.sync_copy(u_hbm.at[pl.ds(wid * _N, _N)], sv_v)
        a2 = scal_v[pl.ds(0, 16)]
        d2 = scal_v[pl.ds(16, 16)]
        iota = jax.lax.broadcasted_iota(jnp.int32, (16,), 0)
        ninf = jnp.full((16,), -jnp.inf, jnp.float32)

        def p1(i, carry):
            m1, m2, m3, m4 = carry
            v = sv_v[pl.ds(i * 16, 16)]
            svv = jnp.maximum(a2 * v + d2, 0.0)
            sv_v[pl.ds(i * 16, 16)] = svv
            lo1 = jnp.minimum(svv, m1)
            m1 = jnp.maximum(svv, m1)
            lo2 = jnp.minimum(lo1, m2)
            m2 = jnp.maximum(lo1, m2)
            lo3 = jnp.minimum(lo2, m3)
            m3 = jnp.maximum(lo2, m3)
            m4 = jnp.maximum(lo3, m4)
            return m1, m2, m3, m4

        tops = jax.lax.fori_loop(0, _N // 16, p1, (ninf, ninf, ninf, ninf))
        tau = jnp.min(tops[3])

        def p2(i, base):
            v = sv_v[pl.ds(i * 16, 16)]
            mask = v >= tau
            c = jnp.sum(mask.astype(jnp.int32))

            @pl.when(c > 0)
            def _():
                pos = plsc.cumsum(mask.astype(jnp.int32)) - 1 + base
                plsc.store_scatter(cand_v, [pos], iota + i * 16, mask=mask)

            return base + c

        ncand = jax.lax.fori_loop(0, _N // 16, p2, jnp.int32(0))
        nvr = (ncand + 15) // 16

        def ext(k, _):
            def scan_vreg(j, carry):
                bv, bi = carry
                idxv = cand_v[pl.ds(j * 16, 16)]
                valid = (j * 16 + iota) < ncand
                vals = plsc.load_gather(sv_v, [jnp.where(valid, idxv, 0)])
                vv = jnp.where(valid, vals, -1.0)
                better = (vv > bv) | ((vv == bv) & (idxv < bi))
                return (jnp.where(better, vv, bv), jnp.where(better, idxv, bi))

            bv, bi = jax.lax.fori_loop(
                0, nvr, scan_vreg,
                (jnp.full((16,), -2.0, jnp.float32),
                 jnp.full((16,), _BIGI, jnp.int32)))
            m = jnp.max(bv)
            best = jnp.min(jnp.where(bv == m, bi, _BIGI))
            lane0 = iota == 0
            ksp = jnp.full((16,), k, jnp.int32)
            plsc.store_scatter(ov_v, [ksp], jnp.full((16,), m, jnp.float32),
                               mask=lane0)
            plsc.store_scatter(oi_v, [ksp], jnp.full((16,), best, jnp.int32),
                               mask=lane0)
            plsc.store_scatter(sv_v, [jnp.full((16,), best, jnp.int32)],
                               jnp.full((16,), -2.0, jnp.float32), mask=lane0)
            return 0

        jax.lax.fori_loop(0, _K, ext, 0)
        pltpu.sync_copy(ov_v, vals_hbm.at[pl.ds(wid * _K, _K)])
        pltpu.sync_copy(oi_v, idx_hbm.at[pl.ds(wid * _K, _K)])


def _topk_kernel(u_ref, gb_ref, vals_ref, idx_ref, work):
    u = u_ref[:, 0, :]
    nn = jnp.float32(_B * _N)
    m2 = jnp.sum(u) / nn
    v2 = jnp.sum(u * u) / nn - m2 * m2
    a2 = gb_ref[0] * jax.lax.rsqrt(v2 + _EPS)
    d2 = gb_ref[1] - a2 * m2
    sv = jnp.maximum(a2 * u + d2, 0.0)
    work[...] = sv

    pos = jax.lax.broadcasted_iota(jnp.int32, (_B, _N), 1)
    kl = jax.lax.broadcasted_iota(jnp.int32, (_B, _K), 1)
    vacc = jnp.zeros((_B, _K), jnp.float32)
    iacc = jnp.zeros((_B, _K), jnp.int32)
    for k in range(_K):
        w = work[...]
        m = jnp.max(w, axis=1, keepdims=True)
        cand = jnp.where(w == m, pos, _BIGI)
        j = jnp.min(cand, axis=1, keepdims=True)
        vacc = vacc + jnp.where(kl == k, m, 0.0)
        iacc = iacc + jnp.where(kl == k, j, 0)
        work[...] = jnp.where(pos == j, -jnp.inf, w)
    vals_ref[...] = vacc
    idx_ref[...] = iacc


def _gather_kernel(idx_sref, in_ref, out_ref, strips, sems):
    b = pl.program_id(0)
    cps = []
    for k in range(_K):
        i = idx_sref[b * _K + k]
        cp = pltpu.make_async_copy(
            in_ref.at[b, :, pl.ds((i // 128) * 128, 128)], strips.at[k],
            sems.at[k])
        cp.start()
        cps.append(cp)
    lane128 = jax.lax.broadcasted_iota(jnp.int32, (_C1, 128), 1)
    kl = jax.lax.broadcasted_iota(jnp.int32, (_C1, _K), 1)
    acc = jnp.zeros((_C1, _K), jnp.float32)
    for k in range(_K):
        cps[k].wait()
        col = jax.lax.rem(idx_sref[b * _K + k], 128)
        v = jnp.sum(jnp.where(lane128 == col, strips[k], 0.0), axis=1)
        acc = acc + jnp.where(kl == k, v[:, None], 0.0)
    out_ref[0] = acc


def _fold_bn(ssum, ssq, g, be, eps=_EPS):
    nn = jnp.float32(_B * _N)
    m = ssum / nn
    v = ssq / nn - m * m
    a = g * jax.lax.rsqrt(v + eps)
    d = be - a * m
    return a, d


@jax.jit
def kernel(sortvec, input, W0, b0, W1, b1, W2, b2, g0, be0, g1, be1, g2, be2):
    s = sortvec[:, 0, :, :]
    w0b = W0[:, 0, :, 0].astype(jnp.bfloat16)
    w1b = W1[:, :, 0, 0].astype(jnp.bfloat16)
    w2b = jnp.zeros((8, _C2), jnp.bfloat16).at[0].set(
        W2[:, :, 0, 0][0].astype(jnp.bfloat16))
    b0c = b0[:, None]
    b1c = b1[:, None]

    seq2 = pltpu.CompilerParams(dimension_semantics=("arbitrary", "arbitrary"))
    nchunks = _N // _CN
    s_spec = pl.BlockSpec((1, _NF, _CN), lambda b, c: (b, 0, c))
    full = lambda shp: pl.BlockSpec(shp, lambda b, c: tuple(0 for _ in shp))

    stats1, sbf = pl.pallas_call(
        _stats_kernel_a,
        grid=(_B, nchunks),
        in_specs=[s_spec, full((_C1, _NF)), full((_C1, 1))],
        out_specs=[full((_C1, 128)), s_spec],
        out_shape=[jax.ShapeDtypeStruct((_C1, 128), jnp.float32),
                   jax.ShapeDtypeStruct((_B, _NF, _N), jnp.bfloat16)],
        compiler_params=seq2,
    )(s, w0b, b0c)
    a0, d0 = _fold_bn(stats1[:, 0], stats1[:, 1], g0, be0)
    aff0 = jnp.stack([a0, d0, b0], axis=1)

    stats2 = pl.pallas_call(
        _stats_kernel_b,
        grid=(_B, nchunks),
        in_specs=[s_spec, full((_C1, _NF)), full((_C1, 3)),
                  full((_C2, _C1)), full((_C2, 1))],
        out_specs=full((_C2, 128)),
        out_shape=jax.ShapeDtypeStruct((_C2, 128), jnp.float32),
        compiler_params=seq2,
    )(sbf, w0b, aff0, w1b, b1c)
    a1, d1 = _fold_bn(stats2[:, 0], stats2[:, 1], g1, be1)
    aff1 = jnp.stack([a1, d1, b1], axis=1)

    u, ustats = pl.pallas_call(
        _score_kernel_c,
        grid=(_B, nchunks),
        in_specs=[s_spec, full((_C1, _NF)), full((_C1, 3)),
                  full((_C2, _C1)), full((_C2, 1)), full((_C2, 3)),
                  full((8, _C2)), pl.BlockSpec(memory_space=pltpu.SMEM)],
        out_specs=[pl.BlockSpec((1, 1, _CN), lambda b, c: (b, 0, c)),
                   full((8, 128))],
        out_shape=[jax.ShapeDtypeStruct((_B, 1, _N), jnp.float32),
                   jax.ShapeDtypeStruct((8, 128), jnp.float32)],
        compiler_params=seq2,
    )(sbf, w0b, aff0, w1b, b1c, aff1, w2b, b2)

    nn = jnp.float32(_B * _N)
    m2 = ustats[0, 0] / nn
    v2 = ustats[1, 0] / nn - m2 * m2
    a2 = g2[0] * jax.lax.rsqrt(v2 + _EPS)
    d2 = be2[0] - a2 * m2
    scal = jnp.concatenate([jnp.full((16,), a2), jnp.full((16,), d2)])

    sel = functools.partial(
        pl.kernel,
        mesh=plsc.VectorSubcoreMesh(core_axis_name="c", subcore_axis_name="s"),
        out_type=[jax.ShapeDtypeStruct((_B * _K,), jnp.float32),
                  jax.ShapeDtypeStruct((_B * _K,), jnp.int32)],
        scratch_types=[pltpu.VMEM((_N,), jnp.float32),
                       pltpu.VMEM((_N,), jnp.int32),
                       pltpu.VMEM((32,), jnp.float32),
                       pltpu.VMEM((_K,), jnp.float32),
                       pltpu.VMEM((_K,), jnp.int32)],
        compiler_params=pltpu.CompilerParams(needs_layout_passes=False),
    )(_sc_select)
    vals_f, idx_f = sel(u.reshape(-1), scal)
    vals = vals_f.reshape(_B, _K)
    idx = idx_f.reshape(_B, _K)

    grid_spec = pltpu.PrefetchScalarGridSpec(
        num_scalar_prefetch=1,
        grid=(_B,),
        in_specs=[pl.BlockSpec(memory_space=pltpu.MemorySpace.HBM)],
        out_specs=pl.BlockSpec((1, _C1, _K), lambda b, iref: (b, 0, 0)),
        scratch_shapes=[pltpu.VMEM((_K, _C1, 128), jnp.float32),
                        pltpu.SemaphoreType.DMA((_K,))],
    )
    sorted_input = pl.pallas_call(
        _gather_kernel,
        grid_spec=grid_spec,
        out_shape=jax.ShapeDtypeStruct((_B, _C1, _K), jnp.float32),
    )(idx.reshape(-1), input)

    feat = jnp.concatenate([sorted_input, vals[:, None, :]], axis=1)
    return (feat, idx)


# speedup vs baseline: 1.0262x; 1.0262x over previous
import functools

import jax
import jax.numpy as jnp
from jax.experimental import pallas as pl
from jax.experimental.pallas import tpu as pltpu
from jax.experimental.pallas import tpu_sc as plsc

_B = 16
_N = 32768
_NF = 32
_C1 = 64
_C2 = 16
_K = 64
_CN = 4096
_EPS = 1e-5
_BIGI = 2 ** 30


def _first_step():
    return (pl.program_id(0) == 0) & (pl.program_id(1) == 0)


def _stats_cols(x):
    ssum = jnp.sum(x, axis=1, keepdims=True)
    ssq = jnp.sum(x * x, axis=1, keepdims=True)
    lane = jax.lax.broadcasted_iota(jnp.int32, (x.shape[0], 128), 1)
    return jnp.where(lane == 0, ssum, 0.0) + jnp.where(lane == 1, ssq, 0.0)


def _mm(w, x):
    return jax.lax.dot_general(w, x, (((1,), (0,)), ((), ())),
                               preferred_element_type=jnp.float32)


def _stats_kernel_a(s_ref, w_ref, b_ref, out_ref, sbf_ref):
    sb = s_ref[0].astype(jnp.bfloat16)
    sbf_ref[0] = sb
    x = _mm(w_ref[...], sb) + b_ref[...]

    @pl.when(_first_step())
    def _():
        out_ref[...] = jnp.zeros_like(out_ref)

    out_ref[...] += _stats_cols(x)


def _layer12(s_ref, w0_ref, aff0_ref, w1_ref, b1_ref):
    x0 = _mm(w0_ref[...], s_ref[0]) + aff0_ref[:, 2:3]
    x1 = jnp.maximum(aff0_ref[:, 0:1] * x0 + aff0_ref[:, 1:2], 0.0)
    return _mm(w1_ref[...], x1.astype(jnp.bfloat16)) + b1_ref[...]


def _stats_kernel_b(s_ref, w0_ref, aff0_ref, w1_ref, b1_ref, out_ref):
    t = _layer12(s_ref, w0_ref, aff0_ref, w1_ref, b1_ref)

    @pl.when(_first_step())
    def _():
        out_ref[...] = jnp.zeros_like(out_ref)

    out_ref[...] += _stats_cols(t)


def _score_kernel_c(s_ref, w0_ref, aff0_ref, w1_ref, b1_ref, aff1_ref,
                    w2_ref, b2_ref, u_ref, us_ref):
    t = _layer12(s_ref, w0_ref, aff0_ref, w1_ref, b1_ref)
    x2 = jnp.maximum(aff1_ref[:, 0:1] * t + aff1_ref[:, 1:2], 0.0)
    u = _mm(w2_ref[...], x2.astype(jnp.bfloat16))
    uf = u[0] + b2_ref[0]
    u_ref[0, 0, :] = uf
    r_io = jax.lax.broadcasted_iota(jnp.int32, (8, 128), 0)
    l_io = jax.lax.broadcasted_iota(jnp.int32, (8, 128), 1)
    contrib = (jnp.where((r_io == 0) & (l_io == 0), jnp.sum(uf), 0.0)
               + jnp.where((r_io == 1) & (l_io == 0), jnp.sum(uf * uf), 0.0))

    @pl.when(_first_step())
    def _():
        us_ref[...] = jnp.zeros_like(us_ref)

    us_ref[...] += contrib


def _sc_select(u_hbm, scal_hbm, vals_hbm, idx_hbm, sv_v, cand_v, scal_v,
               ov_v, oi_v):
    wid = jax.lax.axis_index("s") * 2 + jax.lax.axis_index("c")

    @pl.when(wid < _B)
    def _():
        pltpu.sync_copy(scal_hbm, scal_v)
        pltpu.sync_copy(u_hbm.at[pl.ds(wid * _N, _N)], sv_v)
        a2 = scal_v[pl.ds(0, 16)]
        d2 = scal_v[pl.ds(16, 16)]
        iota = jax.lax.broadcasted_iota(jnp.int32, (16,), 0)
        ninf = jnp.full((16,), -jnp.inf, jnp.float32)

        def p1(i, carry):
            m1, m2, m3, m4 = carry
            v = sv_v[pl.ds(i * 16, 16)]
            svv = jnp.maximum(a2 * v + d2, 0.0)
            sv_v[pl.ds(i * 16, 16)] = svv
            lo1 = jnp.minimum(svv, m1)
            m1 = jnp.maximum(svv, m1)
            lo2 = jnp.minimum(lo1, m2)
            m2 = jnp.maximum(lo1, m2)
            lo3 = jnp.minimum(lo2, m3)
            m3 = jnp.maximum(lo2, m3)
            m4 = jnp.maximum(lo3, m4)
            return m1, m2, m3, m4

        tops = jax.lax.fori_loop(0, _N // 16, p1, (ninf, ninf, ninf, ninf), unroll=8)
        tau = jnp.min(tops[3])

        def p2(i, base):
            v = sv_v[pl.ds(i * 16, 16)]
            mask = v >= tau
            c = jnp.sum(mask.astype(jnp.int32))

            @pl.when(c > 0)
            def _():
                pos = plsc.cumsum(mask.astype(jnp.int32)) - 1 + base
                plsc.store_scatter(cand_v, [pos], iota + i * 16, mask=mask)

            return base + c

        ncand = jax.lax.fori_loop(0, _N // 16, p2, jnp.int32(0), unroll=8)
        nvr = (ncand + 15) // 16

        def ext(k, _):
            def scan_vreg(j, carry):
                bv, bi = carry
                idxv = cand_v[pl.ds(j * 16, 16)]
                valid = (j * 16 + iota) < ncand
                vals = plsc.load_gather(sv_v, [jnp.where(valid, idxv, 0)])
                vv = jnp.where(valid, vals, -1.0)
                better = (vv > bv) | ((vv == bv) & (idxv < bi))
                return (jnp.where(better, vv, bv), jnp.where(better, idxv, bi))

            bv, bi = jax.lax.fori_loop(
                0, nvr, scan_vreg,
                (jnp.full((16,), -2.0, jnp.float32),
                 jnp.full((16,), _BIGI, jnp.int32)))
            m = jnp.max(bv)
            best = jnp.min(jnp.where(bv == m, bi, _BIGI))
            lane0 = iota == 0
            ksp = jnp.full((16,), k, jnp.int32)
            plsc.store_scatter(ov_v, [ksp], jnp.full((16,), m, jnp.float32),
                               mask=lane0)
            plsc.store_scatter(oi_v, [ksp], jnp.full((16,), best, jnp.int32),
                               mask=lane0)
            plsc.store_scatter(sv_v, [jnp.full((16,), best, jnp.int32)],
                               jnp.full((16,), -2.0, jnp.float32), mask=lane0)
            return 0

        jax.lax.fori_loop(0, _K, ext, 0)
        pltpu.sync_copy(ov_v, vals_hbm.at[pl.ds(wid * _K, _K)])
        pltpu.sync_copy(oi_v, idx_hbm.at[pl.ds(wid * _K, _K)])


def _topk_kernel(u_ref, gb_ref, vals_ref, idx_ref, work):
    u = u_ref[:, 0, :]
    nn = jnp.float32(_B * _N)
    m2 = jnp.sum(u) / nn
    v2 = jnp.sum(u * u) / nn - m2 * m2
    a2 = gb_ref[0] * jax.lax.rsqrt(v2 + _EPS)
    d2 = gb_ref[1] - a2 * m2
    sv = jnp.maximum(a2 * u + d2, 0.0)
    work[...] = sv

    pos = jax.lax.broadcasted_iota(jnp.int32, (_B, _N), 1)
    kl = jax.lax.broadcasted_iota(jnp.int32, (_B, _K), 1)
    vacc = jnp.zeros((_B, _K), jnp.float32)
    iacc = jnp.zeros((_B, _K), jnp.int32)
    for k in range(_K):
        w = work[...]
        m = jnp.max(w, axis=1, keepdims=True)
        cand = jnp.where(w == m, pos, _BIGI)
        j = jnp.min(cand, axis=1, keepdims=True)
        vacc = vacc + jnp.where(kl == k, m, 0.0)
        iacc = iacc + jnp.where(kl == k, j, 0)
        work[...] = jnp.where(pos == j, -jnp.inf, w)
    vals_ref[...] = vacc
    idx_ref[...] = iacc


def _gather_kernel(idx_sref, in_ref, out_ref, strips, sems):
    b = pl.program_id(0)
    cps = []
    for k in range(_K):
        i = idx_sref[b * _K + k]
        cp = pltpu.make_async_copy(
            in_ref.at[b, :, pl.ds((i // 128) * 128, 128)], strips.at[k],
            sems.at[k])
        cp.start()
        cps.append(cp)
    lane128 = jax.lax.broadcasted_iota(jnp.int32, (_C1, 128), 1)
    kl = jax.lax.broadcasted_iota(jnp.int32, (_C1, _K), 1)
    acc = jnp.zeros((_C1, _K), jnp.float32)
    for k in range(_K):
        cps[k].wait()
        col = jax.lax.rem(idx_sref[b * _K + k], 128)
        v = jnp.sum(jnp.where(lane128 == col, strips[k], 0.0), axis=1)
        acc = acc + jnp.where(kl == k, v[:, None], 0.0)
    out_ref[0] = acc


def _fold_bn(ssum, ssq, g, be, eps=_EPS):
    nn = jnp.float32(_B * _N)
    m = ssum / nn
    v = ssq / nn - m * m
    a = g * jax.lax.rsqrt(v + eps)
    d = be - a * m
    return a, d


@jax.jit
def kernel(sortvec, input, W0, b0, W1, b1, W2, b2, g0, be0, g1, be1, g2, be2):
    s = sortvec[:, 0, :, :]
    w0b = W0[:, 0, :, 0].astype(jnp.bfloat16)
    w1b = W1[:, :, 0, 0].astype(jnp.bfloat16)
    w2b = jnp.zeros((8, _C2), jnp.bfloat16).at[0].set(
        W2[:, :, 0, 0][0].astype(jnp.bfloat16))
    b0c = b0[:, None]
    b1c = b1[:, None]

    seq2 = pltpu.CompilerParams(dimension_semantics=("arbitrary", "arbitrary"))
    nchunks = _N // _CN
    s_spec = pl.BlockSpec((1, _NF, _CN), lambda b, c: (b, 0, c))
    full = lambda shp: pl.BlockSpec(shp, lambda b, c: tuple(0 for _ in shp))

    stats1, sbf = pl.pallas_call(
        _stats_kernel_a,
        grid=(_B, nchunks),
        in_specs=[s_spec, full((_C1, _NF)), full((_C1, 1))],
        out_specs=[full((_C1, 128)), s_spec],
        out_shape=[jax.ShapeDtypeStruct((_C1, 128), jnp.float32),
                   jax.ShapeDtypeStruct((_B, _NF, _N), jnp.bfloat16)],
        compiler_params=seq2,
    )(s, w0b, b0c)
    a0, d0 = _fold_bn(stats1[:, 0], stats1[:, 1], g0, be0)
    aff0 = jnp.stack([a0, d0, b0], axis=1)

    stats2 = pl.pallas_call(
        _stats_kernel_b,
        grid=(_B, nchunks),
        in_specs=[s_spec, full((_C1, _NF)), full((_C1, 3)),
                  full((_C2, _C1)), full((_C2, 1))],
        out_specs=full((_C2, 128)),
        out_shape=jax.ShapeDtypeStruct((_C2, 128), jnp.float32),
        compiler_params=seq2,
    )(sbf, w0b, aff0, w1b, b1c)
    a1, d1 = _fold_bn(stats2[:, 0], stats2[:, 1], g1, be1)
    aff1 = jnp.stack([a1, d1, b1], axis=1)

    u, ustats = pl.pallas_call(
        _score_kernel_c,
        grid=(_B, nchunks),
        in_specs=[s_spec, full((_C1, _NF)), full((_C1, 3)),
                  full((_C2, _C1)), full((_C2, 1)), full((_C2, 3)),
                  full((8, _C2)), pl.BlockSpec(memory_space=pltpu.SMEM)],
        out_specs=[pl.BlockSpec((1, 1, _CN), lambda b, c: (b, 0, c)),
                   full((8, 128))],
        out_shape=[jax.ShapeDtypeStruct((_B, 1, _N), jnp.float32),
                   jax.ShapeDtypeStruct((8, 128), jnp.float32)],
        compiler_params=seq2,
    )(sbf, w0b, aff0, w1b, b1c, aff1, w2b, b2)

    nn = jnp.float32(_B * _N)
    m2 = ustats[0, 0] / nn
    v2 = ustats[1, 0] / nn - m2 * m2
    a2 = g2[0] * jax.lax.rsqrt(v2 + _EPS)
    d2 = be2[0] - a2 * m2
    scal = jnp.concatenate([jnp.full((16,), a2), jnp.full((16,), d2)])

    sel = functools.partial(
        pl.kernel,
        mesh=plsc.VectorSubcoreMesh(core_axis_name="c", subcore_axis_name="s"),
        out_type=[jax.ShapeDtypeStruct((_B * _K,), jnp.float32),
                  jax.ShapeDtypeStruct((_B * _K,), jnp.int32)],
        scratch_types=[pltpu.VMEM((_N,), jnp.float32),
                       pltpu.VMEM((_N,), jnp.int32),
                       pltpu.VMEM((32,), jnp.float32),
                       pltpu.VMEM((_K,), jnp.float32),
                       pltpu.VMEM((_K,), jnp.int32)],
        compiler_params=pltpu.CompilerParams(needs_layout_passes=False),
    )(_sc_select)
    vals_f, idx_f = sel(u.reshape(-1), scal)
    vals = vals_f.reshape(_B, _K)
    idx = idx_f.reshape(_B, _K)

    grid_spec = pltpu.PrefetchScalarGridSpec(
        num_scalar_prefetch=1,
        grid=(_B,),
        in_specs=[pl.BlockSpec(memory_space=pltpu.MemorySpace.HBM)],
        out_specs=pl.BlockSpec((1, _C1, _K), lambda b, iref: (b, 0, 0)),
        scratch_shapes=[pltpu.VMEM((_K, _C1, 128), jnp.float32),
                        pltpu.SemaphoreType.DMA((_K,))],
    )
    sorted_input = pl.pallas_call(
        _gather_kernel,
        grid_spec=grid_spec,
        out_shape=jax.ShapeDtypeStruct((_B, _C1, _K), jnp.float32),
    )(idx.reshape(-1), input)

    feat = jnp.concatenate([sorted_input, vals[:, None, :]], axis=1)
    return (feat, idx)
